# Initial kernel scaffold; baseline (speedup 1.0000x reference)
import jax
import jax.numpy as jnp
from jax.experimental import pallas as pl


def kernel(x, t, edge_index, edge_attr, bos_mask, rotate_mat, params):
    raise NotImplementedError("write your pallas kernel here")


# speedup vs baseline: 5.7342x; 5.7342x over previous
import functools

import jax
import jax.numpy as jnp
from jax import lax
from jax.experimental import pallas as pl
from jax.experimental.pallas import tpu as pltpu

T_H = 20
EMB = 64
HEADS = 8
DH = EMB // HEADS
_EPS = 1e-5


def _ln(x, w, b):
    mu = jnp.mean(x, axis=-1, keepdims=True)
    var = jnp.mean((x - mu) ** 2, axis=-1, keepdims=True)
    return (x - mu) / jnp.sqrt(var + _EPS) * w + b


def _node_kernel(sc_ref, bost_ref, w0_ref, b0_ref, ln1_ref,
                 w3_ref, b3_ref, ln4_ref, w6_ref, b6_ref, ln7_ref,
                 n1_ref, qw_ref, qb_ref,
                 ce1_ref, center_ref, q_ref):
    x0 = sc_ref[:, 0:1]
    x1 = sc_ref[:, 1:2]
    c = sc_ref[:, 2:3]
    s = sc_ref[:, 3:4]
    bos = sc_ref[:, 4:5]
    ce0 = x0 * c + x1 * s
    ce1c = -x0 * s + x1 * c
    h = ce0 * w0_ref[0:1, :] + ce1c * w0_ref[1:2, :] + b0_ref[0:1, :]
    h = jax.nn.relu(_ln(h, ln1_ref[0:1, :], ln1_ref[1:2, :]))
    h = jnp.dot(h, w3_ref[...], preferred_element_type=jnp.float32) + b3_ref[0:1, :]
    h = jax.nn.relu(_ln(h, ln4_ref[0:1, :], ln4_ref[1:2, :]))
    h = jnp.dot(h, w6_ref[...], preferred_element_type=jnp.float32) + b6_ref[0:1, :]
    h = _ln(h, ln7_ref[0:1, :], ln7_ref[1:2, :])
    ce1 = jnp.where(bos > 0.5, bost_ref[...], h)
    center = _ln(ce1, n1_ref[0:1, :], n1_ref[1:2, :])
    ce1_ref[...] = ce1
    center_ref[...] = center
    q_ref[...] = jnp.dot(center, qw_ref[...], preferred_element_type=jnp.float32) + qb_ref[0:1, :]


def _node_stage(scal, bost, p):
    TN = scal.shape[0]
    R = 2000
    grid = (TN // R,)
    row = lambda i: (i, 0)
    whole = lambda i: (0, 0)
    wspec = pl.BlockSpec((2, EMB), whole)
    vspec = pl.BlockSpec((1, EMB), whole)
    mspec = pl.BlockSpec((EMB, EMB), whole)
    out = pl.pallas_call(
        _node_kernel,
        grid=grid,
        in_specs=[pl.BlockSpec((R, 8), row), pl.BlockSpec((R, EMB), row),
                  wspec, vspec, wspec,
                  mspec, vspec, wspec, mspec, vspec, wspec,
                  wspec, mspec, vspec],
        out_specs=[pl.BlockSpec((R, EMB), row)] * 3,
        out_shape=[jax.ShapeDtypeStruct((TN, EMB), jnp.float32)] * 3,
    )(scal, bost,
      p['ce_w0'].T, p['ce_b0'][None], jnp.stack([p['ce_ln1w'], p['ce_ln1b']]),
      p['ce_w3'].T, p['ce_b3'][None], jnp.stack([p['ce_ln4w'], p['ce_ln4b']]),
      p['ce_w6'].T, p['ce_b6'][None], jnp.stack([p['ce_ln7w'], p['ce_ln7b']]),
      jnp.stack([p['norm1_w'], p['norm1_b']]),
      p['q_w'].T, p['q_b'][None])
    return out


def _edge_kernel(xjr0_ref, xjr1_ref, ear0_ref, ear1_ref, qe_ref,
                 m0w1_ref, m0b1_ref, m0ln_ref, m0w2_ref, m0b2_ref,
                 m1w1_ref, m1b1_ref, m1ln_ref, m1w2_ref, m1b2_ref,
                 aln1_ref, aw_ref, ab_ref, aln2_ref,
                 kw_ref, kb_ref, vw_ref, vb_ref,
                 hsel_ref, hpad_ref, hexp_ref,
                 w_ref, ex_ref):
    f32 = jnp.float32
    h1 = xjr0_ref[...] * m0w1_ref[0:1, :] + xjr1_ref[...] * m0w1_ref[1:2, :] + m0b1_ref[0:1, :]
    h1 = jax.nn.relu(_ln(h1, m0ln_ref[0:1, :], m0ln_ref[1:2, :]))
    e1 = jnp.dot(h1, m0w2_ref[...], preferred_element_type=f32) + m0b2_ref[0:1, :]
    h2 = ear0_ref[...] * m1w1_ref[0:1, :] + ear1_ref[...] * m1w1_ref[1:2, :] + m1b1_ref[0:1, :]
    h2 = jax.nn.relu(_ln(h2, m1ln_ref[0:1, :], m1ln_ref[1:2, :]))
    e2 = jnp.dot(h2, m1w2_ref[...], preferred_element_type=f32) + m1b2_ref[0:1, :]
    z = jax.nn.relu(_ln(e1 + e2, aln1_ref[0:1, :], aln1_ref[1:2, :]))
    nbr = _ln(jnp.dot(z, aw_ref[...], preferred_element_type=f32) + ab_ref[0:1, :],
              aln2_ref[0:1, :], aln2_ref[1:2, :])
    k = jnp.dot(nbr, kw_ref[...], preferred_element_type=f32) + kb_ref[0:1, :]
    v = jnp.dot(nbr, vw_ref[...], preferred_element_type=f32) + vb_ref[0:1, :]
    p = qe_ref[...] * k
    a8 = jnp.dot(p, hsel_ref[...], preferred_element_type=f32) * (1.0 / (DH ** 0.5))
    ex8 = jnp.exp(a8)
    w_ref[...] = v * jnp.dot(ex8, hexp_ref[...], preferred_element_type=f32)
    ex_ref[...] = jnp.dot(ex8, hpad_ref[...], preferred_element_type=f32)


def _edge_stage(xjr0, xjr1, ear0, ear1, qe, p):
    E = qe.shape[0]
    EB = 3200
    grid = (E // EB,)
    row = lambda i: (i, 0)
    whole = lambda i: (0, 0)
    wspec = pl.BlockSpec((2, EMB), whole)
    vspec = pl.BlockSpec((1, EMB), whole)
    mspec = pl.BlockSpec((EMB, EMB), whole)
    cspec = pl.BlockSpec((EB, 1), row)
    hsel = jnp.zeros((EMB, HEADS), jnp.float32)
    hsel = hsel.at[jnp.arange(EMB), jnp.arange(EMB) // DH].set(1.0)
    hpad = jnp.zeros((HEADS, 16), jnp.float32)
    hpad = hpad.at[jnp.arange(HEADS), jnp.arange(HEADS)].set(1.0)
    hexp = hsel.T
    w, ex = pl.pallas_call(
        _edge_kernel,
        grid=grid,
        in_specs=[cspec, cspec, cspec, cspec, pl.BlockSpec((EB, EMB), row),
                  wspec, vspec, wspec, mspec, vspec,
                  wspec, vspec, wspec, mspec, vspec,
                  wspec, mspec, vspec, wspec,
                  mspec, vspec, mspec, vspec,
                  pl.BlockSpec((EMB, HEADS), whole),
                  pl.BlockSpec((HEADS, 16), whole),
                  pl.BlockSpec((HEADS, EMB), whole)],
        out_specs=[pl.BlockSpec((EB, EMB), row), pl.BlockSpec((EB, 16), row)],
        out_shape=[jax.ShapeDtypeStruct((E, EMB), jnp.float32),
                   jax.ShapeDtypeStruct((E, 16), jnp.float32)],
    )(xjr0, xjr1, ear0, ear1, qe,
      p['m0_w1'].T, p['m0_b1'][None], jnp.stack([p['m0_lnw'], p['m0_lnb']]),
      p['m0_w2'].T, p['m0_b2'][None],
      p['m1_w1'].T, p['m1_b1'][None], jnp.stack([p['m1_lnw'], p['m1_lnb']]),
      p['m1_w2'].T, p['m1_b2'][None],
      jnp.stack([p['a_ln1w'], p['a_ln1b']]), p['a_w'].T, p['a_b'][None],
      jnp.stack([p['a_ln2w'], p['a_ln2b']]),
      p['k_w'].T, p['k_b'][None], p['v_w'].T, p['v_b'][None],
      hsel, hpad, hexp)
    return w, ex


def _final_kernel(num_ref, den_ref, ce1_ref, center_ref,
                  ihw_ref, ihb_ref, hhw_ref, hhb_ref, sw_ref, sb_ref,
                  opw_ref, opb_ref, n2_ref, m1w_ref, m1b_ref, m2w_ref, m2b_ref,
                  hexp_ref, out_ref):
    f32 = jnp.float32
    den8 = den_ref[:, 0:HEADS]
    den64 = jnp.dot(den8, hexp_ref[...], preferred_element_type=f32)
    out = num_ref[...] / (den64 + 1e-16)
    center = center_ref[...]
    gate = jax.nn.sigmoid(
        jnp.dot(out, ihw_ref[...], preferred_element_type=f32) + ihb_ref[0:1, :]
        + jnp.dot(center, hhw_ref[...], preferred_element_type=f32) + hhb_ref[0:1, :])
    selfp = jnp.dot(center, sw_ref[...], preferred_element_type=f32) + sb_ref[0:1, :]
    out = out + gate * (selfp - out)
    ce2 = ce1_ref[...] + jnp.dot(out, opw_ref[...], preferred_element_type=f32) + opb_ref[0:1, :]
    h2 = _ln(ce2, n2_ref[0:1, :], n2_ref[1:2, :])
    mlp = jax.nn.relu(jnp.dot(h2, m1w_ref[...], preferred_element_type=f32) + m1b_ref[0:1, :])
    mlp = jnp.dot(mlp, m2w_ref[...], preferred_element_type=f32) + m2b_ref[0:1, :]
    out_ref[...] = ce2 + mlp


def _final_stage(num, den, ce1, center, p):
    TN = num.shape[0]
    R = 2000
    grid = (TN // R,)
    row = lambda i: (i, 0)
    whole = lambda i: (0, 0)
    wspec = pl.BlockSpec((2, EMB), whole)
    vspec = pl.BlockSpec((1, EMB), whole)
    mspec = pl.BlockSpec((EMB, EMB), whole)
    hsel = jnp.zeros((EMB, HEADS), jnp.float32)
    hsel = hsel.at[jnp.arange(EMB), jnp.arange(EMB) // DH].set(1.0)
    hexp = hsel.T
    return pl.pallas_call(
        _final_kernel,
        grid=grid,
        in_specs=[pl.BlockSpec((R, EMB), row), pl.BlockSpec((R, 16), row),
                  pl.BlockSpec((R, EMB), row), pl.BlockSpec((R, EMB), row),
                  mspec, vspec, mspec, vspec, mspec, vspec,
                  mspec, vspec, wspec,
                  pl.BlockSpec((EMB, 4 * EMB), whole), pl.BlockSpec((1, 4 * EMB), whole),
                  pl.BlockSpec((4 * EMB, EMB), whole), vspec,
                  pl.BlockSpec((HEADS, EMB), whole)],
        out_specs=pl.BlockSpec((R, EMB), row),
        out_shape=jax.ShapeDtypeStruct((TN, EMB), jnp.float32),
    )(num, den, ce1, center,
      p['ih_w'].T, p['ih_b'][None], p['hh_w'].T, p['hh_b'][None],
      p['self_w'].T, p['self_b'][None],
      p['op_w'].T, p['op_b'][None], jnp.stack([p['norm2_w'], p['norm2_b']]),
      p['mlp_w1'].T, p['mlp_b1'][None], p['mlp_w2'].T, p['mlp_b2'][None],
      hexp)


def kernel(x, t, edge_index, edge_attr, bos_mask, rotate_mat, params):
    p = params
    TN = x.shape[0]
    N = TN // T_H
    E = edge_index.shape[1]
    dst = edge_index[0]
    src = edge_index[1]

    cth = jnp.tile(rotate_mat[:, 0, 0], T_H)
    sth = jnp.tile(rotate_mat[:, 1, 0], T_H)
    scal = jnp.stack([x[:, 0], x[:, 1], cth, sth,
                      bos_mask.T.reshape(TN).astype(jnp.float32)], axis=1)
    scal = jnp.concatenate([scal, jnp.zeros((TN, 3), jnp.float32)], axis=1)
    bost = jnp.repeat(p['bos_token'], N, axis=0)

    ce1, center, q_all = _node_stage(scal, bost, p)

    xs = x[src]
    ce = cth[dst]
    se = sth[dst]
    xjr0 = xs[:, 0] * ce + xs[:, 1] * se
    xjr1 = -xs[:, 0] * se + xs[:, 1] * ce
    ear0 = edge_attr[:, 0] * ce + edge_attr[:, 1] * se
    ear1 = -edge_attr[:, 0] * se + edge_attr[:, 1] * ce
    qe = q_all[dst]

    w, ex = _edge_stage(xjr0[:, None], xjr1[:, None], ear0[:, None], ear1[:, None], qe, p)

    num = jax.ops.segment_sum(w, dst, num_segments=TN)
    den = jax.ops.segment_sum(ex, dst, num_segments=TN)

    return _final_stage(num, den, ce1, center, p)


# speedup vs baseline: 20.5088x; 3.5766x over previous
import functools

import jax
import jax.numpy as jnp
from jax import lax
from jax.experimental import pallas as pl
from jax.experimental.pallas import tpu as pltpu
from jax.experimental.pallas import tpu_sc as plsc

T_H = 20
EMB = 64
HEADS = 8
DH = EMB // HEADS
_EPS = 1e-5

NC = 2
NS = 16
NW = NC * NS
LANES = 16
CH = 128


def _ln(x, w, b):
    mu = jnp.mean(x, axis=-1, keepdims=True)
    var = jnp.mean((x - mu) ** 2, axis=-1, keepdims=True)
    return (x - mu) / jnp.sqrt(var + _EPS) * w + b


def _node_kernel(sc_ref, bost_ref, w0_ref, b0_ref, ln1_ref,
                 w3_ref, b3_ref, ln4_ref, w6_ref, b6_ref, ln7_ref,
                 n1_ref, qw_ref, qb_ref,
                 ce1_ref, center_ref, q_ref):
    x0 = sc_ref[:, 0:1]
    x1 = sc_ref[:, 1:2]
    c = sc_ref[:, 2:3]
    s = sc_ref[:, 3:4]
    bos = sc_ref[:, 4:5]
    ce0 = x0 * c + x1 * s
    ce1c = -x0 * s + x1 * c
    h = ce0 * w0_ref[0:1, :] + ce1c * w0_ref[1:2, :] + b0_ref[0:1, :]
    h = jax.nn.relu(_ln(h, ln1_ref[0:1, :], ln1_ref[1:2, :]))
    h = jnp.dot(h, w3_ref[...], preferred_element_type=jnp.float32) + b3_ref[0:1, :]
    h = jax.nn.relu(_ln(h, ln4_ref[0:1, :], ln4_ref[1:2, :]))
    h = jnp.dot(h, w6_ref[...], preferred_element_type=jnp.float32) + b6_ref[0:1, :]
    h = _ln(h, ln7_ref[0:1, :], ln7_ref[1:2, :])
    ce1 = jnp.where(bos > 0.5, bost_ref[...], h)
    center = _ln(ce1, n1_ref[0:1, :], n1_ref[1:2, :])
    ce1_ref[...] = ce1
    center_ref[...] = center
    q_ref[...] = jnp.dot(center, qw_ref[...], preferred_element_type=jnp.float32) + qb_ref[0:1, :]


def _node_stage(scal, bost, p):
    TN = scal.shape[0]
    R = 2000
    grid = (TN // R,)
    row = lambda i: (i, 0)
    whole = lambda i: (0, 0)
    wspec = pl.BlockSpec((2, EMB), whole)
    vspec = pl.BlockSpec((1, EMB), whole)
    mspec = pl.BlockSpec((EMB, EMB), whole)
    out = pl.pallas_call(
        _node_kernel,
        grid=grid,
        in_specs=[pl.BlockSpec((R, 8), row), pl.BlockSpec((R, EMB), row),
                  wspec, vspec, wspec,
                  mspec, vspec, wspec, mspec, vspec, wspec,
                  wspec, mspec, vspec],
        out_specs=[pl.BlockSpec((R, EMB), row)] * 3,
        out_shape=[jax.ShapeDtypeStruct((TN, EMB), jnp.float32)] * 3,
    )(scal, bost,
      p['ce_w0'].T, p['ce_b0'][None], jnp.stack([p['ce_ln1w'], p['ce_ln1b']]),
      p['ce_w3'].T, p['ce_b3'][None], jnp.stack([p['ce_ln4w'], p['ce_ln4b']]),
      p['ce_w6'].T, p['ce_b6'][None], jnp.stack([p['ce_ln7w'], p['ce_ln7b']]),
      jnp.stack([p['norm1_w'], p['norm1_b']]),
      p['q_w'].T, p['q_b'][None])
    return out


def _gather_body(q_hbm, dst_hbm, src_hbm, ea0_hbm, ea1_hbm,
                 x0_hbm, x1_hbm, c_hbm, s_hbm,
                 qe_out, xjr0_out, xjr1_out, ear0_out, ear1_out,
                 x0_v, x1_v, c_v, s_v, dstv, srcv, ea0v, ea1v,
                 xj0v, xj1v, er0v, er1v, qrows, sem):
    cidx = lax.axis_index("c")
    sidx = lax.axis_index("s")
    wid = sidx * NC + cidx
    pltpu.sync_copy(x0_hbm, x0_v)
    pltpu.sync_copy(x1_hbm, x1_v)
    pltpu.sync_copy(c_hbm, c_v)
    pltpu.sync_copy(s_hbm, s_v)
    nchunk = dst_hbm.shape[0] // CH
    nloops = nchunk // NW + jnp.where(wid < nchunk % NW, 1, 0)

    def body(j, carry):
        cid = wid + NW * j
        base = cid * CH
        pltpu.sync_copy(dst_hbm.at[pl.ds(base, CH)], dstv)
        pltpu.sync_copy(src_hbm.at[pl.ds(base, CH)], srcv)
        pltpu.sync_copy(ea0_hbm.at[pl.ds(base, CH)], ea0v)
        pltpu.sync_copy(ea1_hbm.at[pl.ds(base, CH)], ea1v)
        cp = pltpu.async_copy(q_hbm.at[dstv], qrows, sem)
        for b in range(CH // LANES):
            sl = pl.ds(b * LANES, LANES)
            d16 = dstv[sl]
            s16 = srcv[sl]
            xs0 = plsc.load_gather(x0_v, [s16])
            xs1 = plsc.load_gather(x1_v, [s16])
            cc = plsc.load_gather(c_v, [d16])
            ss = plsc.load_gather(s_v, [d16])
            e0 = ea0v[sl]
            e1 = ea1v[sl]
            xj0v[sl] = xs0 * cc + xs1 * ss
            xj1v[sl] = -xs0 * ss + xs1 * cc
            er0v[sl] = e0 * cc + e1 * ss
            er1v[sl] = -e0 * ss + e1 * cc
        cp.wait()
        pltpu.sync_copy(qrows, qe_out.at[pl.ds(base, CH)])
        pltpu.sync_copy(xj0v, xjr0_out.at[pl.ds(base, CH)])
        pltpu.sync_copy(xj1v, xjr1_out.at[pl.ds(base, CH)])
        pltpu.sync_copy(er0v, ear0_out.at[pl.ds(base, CH)])
        pltpu.sync_copy(er1v, ear1_out.at[pl.ds(base, CH)])
        return carry

    lax.fori_loop(0, nloops, body, 0)


def _gather_stage(q_all, dst, src, ea0, ea1, x0, x1, cth, sth):
    E = dst.shape[0]
    TN = q_all.shape[0]
    f32 = jnp.float32
    mesh = plsc.VectorSubcoreMesh(core_axis_name="c", subcore_axis_name="s")
    fn = pl.kernel(
        _gather_body,
        out_type=[jax.ShapeDtypeStruct((E, EMB), f32)]
        + [jax.ShapeDtypeStruct((E,), f32)] * 4,
        mesh=mesh,
        scratch_types=[pltpu.VMEM((TN,), f32)] * 4
        + [pltpu.VMEM((CH,), jnp.int32)] * 2
        + [pltpu.VMEM((CH,), f32)] * 6
        + [pltpu.VMEM((CH, EMB), f32), pltpu.SemaphoreType.DMA],
        compiler_params=pltpu.CompilerParams(needs_layout_passes=False, use_tc_tiling_on_sc=False),
    )
    return fn(q_all, dst, src, ea0, ea1, x0, x1, cth, sth)


def _scatter_body(dst_hbm, w_hbm, ex_hbm, zn_hbm, zd_hbm,
                  nump_out, denp_out,
                  acc_n, acc_d, dstv, wv, exv):
    cidx = lax.axis_index("c")
    sidx = lax.axis_index("s")
    TN = zn_hbm.shape[0] * NS
    srows = TN // NS
    pltpu.sync_copy(zn_hbm, acc_n.at[pl.ds(sidx * srows, srows)])
    pltpu.sync_copy(zd_hbm, acc_d.at[pl.ds(sidx * srows, srows)])
    plsc.subcore_barrier()
    nchunk = dst_hbm.shape[0] // CH
    per_core = nchunk // NC
    nloops = per_core // NS + jnp.where(sidx < per_core % NS, 1, 0)

    def body(m, carry):
        cid = NC * (sidx + NS * m) + cidx
        base = cid * CH
        pltpu.sync_copy(dst_hbm.at[pl.ds(base, CH)], dstv)
        pltpu.sync_copy(w_hbm.at[pl.ds(base, CH)], wv)
        pltpu.sync_copy(ex_hbm.at[pl.ds(base, CH)], exv)
        pltpu.sync_copy(wv, acc_n.at[dstv], add=True)
        pltpu.sync_copy(exv, acc_d.at[dstv], add=True)
        return carry

    lax.fori_loop(0, nloops, body, 0)
    plsc.subcore_barrier()
    sl = pl.ds(sidx * srows, srows)
    pltpu.sync_copy(acc_n.at[sl], nump_out.at[cidx, sl])
    pltpu.sync_copy(acc_d.at[sl], denp_out.at[cidx, sl])


def _scatter_stage(dst, w, ex, TN):
    E = dst.shape[0]
    f32 = jnp.float32
    srows = TN // NS
    mesh = plsc.VectorSubcoreMesh(core_axis_name="c", subcore_axis_name="s")
    fn = pl.kernel(
        _scatter_body,
        out_type=[jax.ShapeDtypeStruct((NC, TN, EMB), f32),
                  jax.ShapeDtypeStruct((NC, TN, 16), f32)],
        mesh=mesh,
        scratch_types=[pltpu.VMEM_SHARED((TN, EMB), f32),
                       pltpu.VMEM_SHARED((TN, 16), f32),
                       pltpu.VMEM((CH,), jnp.int32),
                       pltpu.VMEM((CH, EMB), f32),
                       pltpu.VMEM((CH, 16), f32)],
        compiler_params=pltpu.CompilerParams(needs_layout_passes=False, use_tc_tiling_on_sc=False),
    )
    zn = jnp.zeros((srows, EMB), f32)
    zd = jnp.zeros((srows, 16), f32)
    return fn(dst, w, ex, zn, zd)


def _edge_kernel(xjr0_ref, xjr1_ref, ear0_ref, ear1_ref, qe_ref,
                 m0w1_ref, m0b1_ref, m0ln_ref, m0w2_ref, m0b2_ref,
                 m1w1_ref, m1b1_ref, m1ln_ref, m1w2_ref, m1b2_ref,
                 aln1_ref, aw_ref, ab_ref, aln2_ref,
                 kw_ref, kb_ref, vw_ref, vb_ref,
                 hsel_ref, hpad_ref, hexp_ref,
                 w_ref, ex_ref):
    f32 = jnp.float32
    h1 = xjr0_ref[...] * m0w1_ref[0:1, :] + xjr1_ref[...] * m0w1_ref[1:2, :] + m0b1_ref[0:1, :]
    h1 = jax.nn.relu(_ln(h1, m0ln_ref[0:1, :], m0ln_ref[1:2, :]))
    e1 = jnp.dot(h1, m0w2_ref[...], preferred_element_type=f32) + m0b2_ref[0:1, :]
    h2 = ear0_ref[...] * m1w1_ref[0:1, :] + ear1_ref[...] * m1w1_ref[1:2, :] + m1b1_ref[0:1, :]
    h2 = jax.nn.relu(_ln(h2, m1ln_ref[0:1, :], m1ln_ref[1:2, :]))
    e2 = jnp.dot(h2, m1w2_ref[...], preferred_element_type=f32) + m1b2_ref[0:1, :]
    z = jax.nn.relu(_ln(e1 + e2, aln1_ref[0:1, :], aln1_ref[1:2, :]))
    nbr = _ln(jnp.dot(z, aw_ref[...], preferred_element_type=f32) + ab_ref[0:1, :],
              aln2_ref[0:1, :], aln2_ref[1:2, :])
    k = jnp.dot(nbr, kw_ref[...], preferred_element_type=f32) + kb_ref[0:1, :]
    v = jnp.dot(nbr, vw_ref[...], preferred_element_type=f32) + vb_ref[0:1, :]
    p = qe_ref[...] * k
    a8 = jnp.dot(p, hsel_ref[...], preferred_element_type=f32) * (1.0 / (DH ** 0.5))
    ex8 = jnp.exp(a8)
    w_ref[...] = v * jnp.dot(ex8, hexp_ref[...], preferred_element_type=f32)
    ex_ref[...] = jnp.dot(ex8, hpad_ref[...], preferred_element_type=f32)


def _edge_stage(xjr0, xjr1, ear0, ear1, qe, p):
    E = qe.shape[0]
    EB = 3200
    grid = (E // EB,)
    row = lambda i: (i, 0)
    whole = lambda i: (0, 0)
    wspec = pl.BlockSpec((2, EMB), whole)
    vspec = pl.BlockSpec((1, EMB), whole)
    mspec = pl.BlockSpec((EMB, EMB), whole)
    cspec = pl.BlockSpec((EB, 1), row)
    hsel = jnp.zeros((EMB, HEADS), jnp.float32)
    hsel = hsel.at[jnp.arange(EMB), jnp.arange(EMB) // DH].set(1.0)
    hpad = jnp.zeros((HEADS, 16), jnp.float32)
    hpad = hpad.at[jnp.arange(HEADS), jnp.arange(HEADS)].set(1.0)
    hexp = hsel.T
    w, ex = pl.pallas_call(
        _edge_kernel,
        grid=grid,
        in_specs=[cspec, cspec, cspec, cspec, pl.BlockSpec((EB, EMB), row),
                  wspec, vspec, wspec, mspec, vspec,
                  wspec, vspec, wspec, mspec, vspec,
                  wspec, mspec, vspec, wspec,
                  mspec, vspec, mspec, vspec,
                  pl.BlockSpec((EMB, HEADS), whole),
                  pl.BlockSpec((HEADS, 16), whole),
                  pl.BlockSpec((HEADS, EMB), whole)],
        out_specs=[pl.BlockSpec((EB, EMB), row), pl.BlockSpec((EB, 16), row)],
        out_shape=[jax.ShapeDtypeStruct((E, EMB), jnp.float32),
                   jax.ShapeDtypeStruct((E, 16), jnp.float32)],
    )(xjr0, xjr1, ear0, ear1, qe,
      p['m0_w1'].T, p['m0_b1'][None], jnp.stack([p['m0_lnw'], p['m0_lnb']]),
      p['m0_w2'].T, p['m0_b2'][None],
      p['m1_w1'].T, p['m1_b1'][None], jnp.stack([p['m1_lnw'], p['m1_lnb']]),
      p['m1_w2'].T, p['m1_b2'][None],
      jnp.stack([p['a_ln1w'], p['a_ln1b']]), p['a_w'].T, p['a_b'][None],
      jnp.stack([p['a_ln2w'], p['a_ln2b']]),
      p['k_w'].T, p['k_b'][None], p['v_w'].T, p['v_b'][None],
      hsel, hpad, hexp)
    return w, ex


def _final_kernel(num_ref, den_ref, ce1_ref, center_ref,
                  ihw_ref, ihb_ref, hhw_ref, hhb_ref, sw_ref, sb_ref,
                  opw_ref, opb_ref, n2_ref, m1w_ref, m1b_ref, m2w_ref, m2b_ref,
                  hexp_ref, out_ref):
    f32 = jnp.float32
    den8 = (den_ref[0] + den_ref[1])[:, 0:HEADS]
    den64 = jnp.dot(den8, hexp_ref[...], preferred_element_type=f32)
    out = (num_ref[0] + num_ref[1]) / (den64 + 1e-16)
    center = center_ref[...]
    gate = jax.nn.sigmoid(
        jnp.dot(out, ihw_ref[...], preferred_element_type=f32) + ihb_ref[0:1, :]
        + jnp.dot(center, hhw_ref[...], preferred_element_type=f32) + hhb_ref[0:1, :])
    selfp = jnp.dot(center, sw_ref[...], preferred_element_type=f32) + sb_ref[0:1, :]
    out = out + gate * (selfp - out)
    ce2 = ce1_ref[...] + jnp.dot(out, opw_ref[...], preferred_element_type=f32) + opb_ref[0:1, :]
    h2 = _ln(ce2, n2_ref[0:1, :], n2_ref[1:2, :])
    mlp = jax.nn.relu(jnp.dot(h2, m1w_ref[...], preferred_element_type=f32) + m1b_ref[0:1, :])
    mlp = jnp.dot(mlp, m2w_ref[...], preferred_element_type=f32) + m2b_ref[0:1, :]
    out_ref[...] = ce2 + mlp


def _final_stage(num, den, ce1, center, p):
    TN = num.shape[1]
    R = 2000
    grid = (TN // R,)
    row = lambda i: (i, 0)
    whole = lambda i: (0, 0)
    wspec = pl.BlockSpec((2, EMB), whole)
    vspec = pl.BlockSpec((1, EMB), whole)
    mspec = pl.BlockSpec((EMB, EMB), whole)
    hsel = jnp.zeros((EMB, HEADS), jnp.float32)
    hsel = hsel.at[jnp.arange(EMB), jnp.arange(EMB) // DH].set(1.0)
    hexp = hsel.T
    return pl.pallas_call(
        _final_kernel,
        grid=grid,
        in_specs=[pl.BlockSpec((NC, R, EMB), lambda i: (0, i, 0)),
                  pl.BlockSpec((NC, R, 16), lambda i: (0, i, 0)),
                  pl.BlockSpec((R, EMB), row), pl.BlockSpec((R, EMB), row),
                  mspec, vspec, mspec, vspec, mspec, vspec,
                  mspec, vspec, wspec,
                  pl.BlockSpec((EMB, 4 * EMB), whole), pl.BlockSpec((1, 4 * EMB), whole),
                  pl.BlockSpec((4 * EMB, EMB), whole), vspec,
                  pl.BlockSpec((HEADS, EMB), whole)],
        out_specs=pl.BlockSpec((R, EMB), row),
        out_shape=jax.ShapeDtypeStruct((TN, EMB), jnp.float32),
    )(num, den, ce1, center,
      p['ih_w'].T, p['ih_b'][None], p['hh_w'].T, p['hh_b'][None],
      p['self_w'].T, p['self_b'][None],
      p['op_w'].T, p['op_b'][None], jnp.stack([p['norm2_w'], p['norm2_b']]),
      p['mlp_w1'].T, p['mlp_b1'][None], p['mlp_w2'].T, p['mlp_b2'][None],
      hexp)


def kernel(x, t, edge_index, edge_attr, bos_mask, rotate_mat, params):
    p = params
    TN = x.shape[0]
    N = TN // T_H
    E = edge_index.shape[1]
    dst = edge_index[0]
    src = edge_index[1]

    cth = jnp.tile(rotate_mat[:, 0, 0], T_H)
    sth = jnp.tile(rotate_mat[:, 1, 0], T_H)
    scal = jnp.stack([x[:, 0], x[:, 1], cth, sth,
                      bos_mask.T.reshape(TN).astype(jnp.float32)], axis=1)
    scal = jnp.concatenate([scal, jnp.zeros((TN, 3), jnp.float32)], axis=1)
    bost = jnp.repeat(p['bos_token'], N, axis=0)

    ce1, center, q_all = _node_stage(scal, bost, p)

    qe, xjr0, xjr1, ear0, ear1 = _gather_stage(
        q_all, dst, src, edge_attr[:, 0], edge_attr[:, 1],
        x[:, 0], x[:, 1], cth, sth)

    w, ex = _edge_stage(xjr0[:, None], xjr1[:, None], ear0[:, None], ear1[:, None], qe, p)

    num, den = _scatter_stage(dst, w, ex, TN)

    return _final_stage(num, den, ce1, center, p)
